# baseline (speedup 1.0000x reference)
import functools

import jax
import jax.numpy as jnp
from jax import lax
from jax.experimental import pallas as pl
from jax.experimental.pallas import tpu as pltpu
from jax.experimental.pallas import tpu_sc as plsc

B, S = 4, 2048
N = B * S
H = 768
NUM_EMB = H // 4
UNI_LAYOUT = NUM_EMB // 2
UNI_TREE = (H - NUM_EMB) // 2
EPS = 1e-6

NC, NS, L = 2, 16, 16
NW = NC * NS
PER_W = N // NW
CHUNK = 32
NBUF = 4
NCHUNK = PER_W // CHUNK


def _sc_gather():
    mesh = plsc.VectorSubcoreMesh(core_axis_name="c", subcore_axis_name="s")

    slot = [
        pltpu.VMEM((CHUNK, H), jnp.float32),
        pltpu.SemaphoreType.DMA,
        pltpu.SemaphoreType.DMA,
    ]

    @functools.partial(
        pl.kernel,
        mesh=mesh,
        out_type=jax.ShapeDtypeStruct((N, H), jnp.float32),
        compiler_params=pltpu.CompilerParams(needs_layout_passes=False),
        scratch_types=[pltpu.VMEM((PER_W,), jnp.int32)]
        + slot + slot + slot + slot,
    )
    def sc_kernel(tok_id, tokW, out_hbm, i_tok,
                  b0, g0, s0, b1, g1, s1, b2, g2, s2, b3, g3, s3):
        wid = lax.axis_index("s") * NC + lax.axis_index("c")
        w0 = wid * PER_W
        pltpu.sync_copy(tok_id.at[w0 // S, pl.ds(w0 % S, PER_W)], i_tok)

        bufs = ((b0, g0, s0), (b1, g1, s1), (b2, g2, s2), (b3, g3, s3))

        def gather(c, bset):
            return pltpu.make_async_copy(
                tokW.at[i_tok.at[pl.ds(c * CHUNK, CHUNK)]], bset[0], bset[1])

        def store(c, bset):
            return pltpu.make_async_copy(
                bset[0], out_hbm.at[pl.ds(w0 + c * CHUNK, CHUNK), :], bset[2])

        gather(0, bufs[0]).start()
        gather(1, bufs[1]).start()

        def chunk_body(c, carry):
            for b in range(NBUF):

                @pl.when(c % NBUF == b)
                def _():
                    gather(c, bufs[b]).wait()
                    store(c, bufs[b]).start()

                    b2i = (b + 2) % NBUF

                    @pl.when(c >= 2)
                    def _():
                        store(c - 2, bufs[b2i]).wait()

                    @pl.when(c + 2 < NCHUNK)
                    def _():
                        gather(c + 2, bufs[b2i]).start()

            return carry

        lax.fori_loop(0, NCHUNK, chunk_body, 0)
        store(NCHUNK - 2, bufs[(NCHUNK - 2) % NBUF]).wait()
        store(NCHUNK - 1, bufs[(NCHUNK - 1) % NBUF]).wait()

    return sc_kernel


_SC_GATHER = _sc_gather()

TC_BLK = 1024


def _onehot(ids, n):
    return (ids[:, None]
            == lax.broadcasted_iota(jnp.int32, (TC_BLK, n), 1)
            ).astype(jnp.float32)


def _select(oh, w_ref):
    w = w_ref[...]
    hi = w.astype(jnp.bfloat16).astype(jnp.float32)
    lo = w - hi
    d = functools.partial(jnp.dot, preferred_element_type=jnp.float32)
    return d(oh, hi) + d(oh, lo)


def _tc_body(part_ref, ord_ref, mag_ref, pre_ref, top_ref, low_ref,
             row_ref, col_ref, pt_ref, pl_ref, fv_ref,
             ordW_ref, magW_ref, preW_ref, topW_ref, lowW_ref,
             rowW_ref, colW_ref, treeW_ref, fmtW_ref, g_ref, b_ref, o_ref):
    f32 = jnp.float32
    bi = pl.program_id(0)
    sj = pl.ds(pl.program_id(1) * TC_BLK, TC_BLK)
    numeric = jnp.concatenate(
        [_select(_onehot(mag_ref[bi, sj], 12), magW_ref),
         _select(_onehot(pre_ref[bi, sj], 12), preW_ref),
         _select(_onehot(top_ref[bi, sj], 12), topW_ref),
         _select(_onehot(low_ref[bi, sj], 12), lowW_ref)], axis=1)
    order_states = _select(_onehot(ord_ref[bi, sj], 256), ordW_ref)
    row_states = _select(_onehot(row_ref[bi, sj], 257), rowW_ref)
    col_states = _select(_onehot(col_ref[bi, sj], 257), colW_ref)
    ptf = pt_ref[0].astype(f32)
    plf = pl_ref[0].astype(f32)
    top_tree = jnp.tile(ptf, (1, 3)) * treeW_ref[0][None, :]
    left_tree = jnp.tile(plf, (1, 3)) * treeW_ref[1][None, :]
    position = order_states + jnp.concatenate(
        [row_states, left_tree, col_states, top_tree], axis=1)
    fv = fv_ref[0]
    fv_hi = fv.astype(jnp.bfloat16).astype(f32)
    fv_lo = fv - fv_hi
    fw = fmtW_ref[...]
    fw_hi = fw.astype(jnp.bfloat16).astype(f32)
    fw_lo = fw - fw_hi
    dg = functools.partial(
        lax.dot_general, dimension_numbers=(((1,), (1,)), ((), ())),
        preferred_element_type=f32)
    fmt_states = dg(fv_hi, fw_hi) + dg(fv_hi, fw_lo) + dg(fv_lo, fw_hi)
    x = part_ref[...] + numeric + position + fmt_states
    mean = jnp.mean(x, axis=-1, keepdims=True)
    var = jnp.mean((x - mean) ** 2, axis=-1, keepdims=True)
    o_ref[0] = ((x - mean) * lax.rsqrt(var + EPS) * g_ref[...]
                + b_ref[...])


def _tc_finish(partial, order, num_mag, num_pre, num_top, num_low,
               pos_row, pos_col, pos_top, pos_left, format_vec,
               order_W, mag_W, pre_W, top_W, low_W, row_W, col_W,
               tree_W, fmt_W, ln_g, ln_b):
    grid = (B, S // TC_BLK)
    ids_spec = pl.BlockSpec((B, S), lambda i, j: (0, 0))
    pos_spec = pl.BlockSpec((1, TC_BLK, UNI_LAYOUT), lambda i, j: (i, j, 0))
    full = lambda *shape: pl.BlockSpec(shape, lambda i, j: (0,) * len(shape))
    return pl.pallas_call(
        _tc_body,
        grid=grid,
        in_specs=[
            pl.BlockSpec((TC_BLK, H),
                         lambda i, j: (i * (S // TC_BLK) + j, 0)),
            ids_spec, ids_spec, ids_spec, ids_spec, ids_spec,
            ids_spec, ids_spec, pos_spec, pos_spec,
            pl.BlockSpec((1, TC_BLK, 11), lambda i, j: (i, j, 0)),
            full(256, H), full(12, NUM_EMB), full(12, NUM_EMB),
            full(12, NUM_EMB), full(12, NUM_EMB),
            full(257, UNI_LAYOUT), full(257, UNI_LAYOUT),
            full(2, UNI_TREE), full(H, 11), full(H,), full(H,),
        ],
        out_specs=pl.BlockSpec((1, TC_BLK, H), lambda i, j: (i, j, 0)),
        out_shape=jax.ShapeDtypeStruct((B, S, H), jnp.float32),
    )(partial, order, num_mag, num_pre, num_top, num_low,
      pos_row, pos_col, pos_top, pos_left, format_vec,
      order_W, mag_W, pre_W, top_W, low_W, row_W, col_W,
      tree_W, fmt_W, ln_g, ln_b)


def kernel(token_id, num_mag, num_pre, num_top, num_low, order, pos_row,
           pos_col, pos_top, pos_left, format_vec, token_W, mag_W, pre_W,
           top_W, low_W, order_W, row_W, col_W, tree_W, fmt_W, ln_g, ln_b):
    i32 = jnp.int32
    partial = _SC_GATHER(token_id.astype(i32), token_W)
    return _tc_finish(partial, order.astype(i32), num_mag.astype(i32),
                      num_pre.astype(i32), num_top.astype(i32),
                      num_low.astype(i32), pos_row.astype(i32),
                      pos_col.astype(i32), pos_top.astype(i32),
                      pos_left.astype(i32), format_vec,
                      order_W, mag_W, pre_W, top_W, low_W, row_W, col_W,
                      tree_W, fmt_W, ln_g, ln_b)
